# baseline (speedup 1.0000x reference)
import functools

import jax
import jax.numpy as jnp
import numpy as np
from jax import lax
from jax.experimental import pallas as pl
from jax.experimental.pallas import tpu as pltpu
from jax.experimental.pallas import tpu_sc as plsc

N = 10000
E = 320000
H = 128
KP = 512
BN = 512
NP = 10240
NB = NP // BN

NC = 2
NS = 16
NW = NC * NS
EPW = E // NW
CH = 80
NCHUNK = EPW // CH
assert NCHUNK % 4 == 1
RPW = NP // NS


def _leaky(v):
    return jnp.maximum(v, 0.01 * v)


def _unpack_perm():
    tau = np.empty((H,), np.int64)
    for k in range(H // 32):
        for i in range(16):
            tau[32 * k + i] = 32 * k + 2 * i
            tau[32 * k + 16 + i] = 32 * k + 2 * i + 1
    g = np.empty((H,), np.int64)
    g[tau] = np.arange(H)
    return g


_PERM = _unpack_perm()


def _encode_body(xz_ref, cl_ref, w_enc_ref, b_enc_ref, wmd_ref, bm_ref,
                 wms_ref, h_ref, p_ref, q_ref, csum_ref, ccnt_ref):
    i = pl.program_id(0)
    h = _leaky(jnp.dot(xz_ref[...], w_enc_ref[...],
                       preferred_element_type=jnp.float32) + b_enc_ref[...])
    h_ref[...] = h
    p_ref[...] = (jnp.dot(h, wmd_ref[...], preferred_element_type=jnp.float32)
                  + bm_ref[...]).astype(jnp.bfloat16)
    q_ref[...] = jnp.dot(h, wms_ref[...],
                         preferred_element_type=jnp.float32
                         ).astype(jnp.bfloat16)
    cl = cl_ref[0, 0, :]
    onehot = (cl[:, None] == lax.broadcasted_iota(jnp.int32, (BN, KP), 1))
    onehot = onehot.astype(jnp.float32)

    @pl.when(i == 0)
    def _():
        csum_ref[...] = jnp.zeros((KP, H), jnp.float32)
        ccnt_ref[...] = jnp.zeros((KP, H), jnp.float32)

    csum_ref[...] += lax.dot_general(onehot, h, (((0,), (0,)), ((), ())),
                                     preferred_element_type=jnp.float32)
    ccnt_ref[...] += jnp.sum(onehot, axis=0)[:, None]


def _run_encode(xz, cl3, w_enc, b_enc, wmd, bm, wms):
    whole2 = lambda: pl.BlockSpec(index_map=lambda i: (0, 0))
    return pl.pallas_call(
        _encode_body,
        grid=(NB,),
        in_specs=[
            pl.BlockSpec((BN, xz.shape[1]), lambda i: (i, 0)),
            pl.BlockSpec((1, 1, BN), lambda i: (i, 0, 0)),
            whole2(), whole2(), whole2(), whole2(), whole2(),
        ],
        out_specs=[
            pl.BlockSpec((BN, H), lambda i: (i, 0)),
            pl.BlockSpec((BN, H), lambda i: (i, 0)),
            pl.BlockSpec((BN, H), lambda i: (i, 0)),
            pl.BlockSpec((KP, H), lambda i: (0, 0)),
            pl.BlockSpec((KP, H), lambda i: (0, 0)),
        ],
        out_shape=[
            jax.ShapeDtypeStruct((NP, H), jnp.float32),
            jax.ShapeDtypeStruct((NP, H), jnp.bfloat16),
            jax.ShapeDtypeStruct((NP, H), jnp.bfloat16),
            jax.ShapeDtypeStruct((KP, H), jnp.float32),
            jax.ShapeDtypeStruct((KP, H), jnp.float32),
        ],
    )(xz, cl3, w_enc, b_enc, wmd, bm, wms)


def _update_body(last, h_ref, ap_ref, dp_ref, csum_ref, ccnt_ref, cl_ref,
                 rt_ref, wu_ref, bu_ref, wmd_ref, bm_ref, wms_ref, *outs):
    i = pl.program_id(0)
    h = h_ref[...]
    d3 = dp_ref[...]
    deg = jnp.maximum(d3[0, :, 0] + d3[1, :, 0], 1.0)
    a3 = ap_ref[...]
    aggr = (a3[0] + a3[1]) / deg[:, None]
    cmean = csum_ref[...] / jnp.maximum(ccnt_ref[...], 1.0)
    cl = cl_ref[0, 0, :]
    onehot = (cl[:, None] == lax.broadcasted_iota(jnp.int32, (BN, KP), 1))
    onehot = onehot.astype(jnp.float32)
    c = jnp.dot(onehot, cmean,
                preferred_element_type=jnp.float32) * rt_ref[0, 0, :][:, None]
    wu = wu_ref[...]
    u = (jnp.dot(h, wu[:H], preferred_element_type=jnp.float32)
         + jnp.dot(aggr, wu[H:2 * H], preferred_element_type=jnp.float32)
         + jnp.dot(c, wu[2 * H:], preferred_element_type=jnp.float32)
         + bu_ref[...])
    hn = h + _leaky(u)
    outs[0][...] = hn
    if not last:
        outs[1][...] = (jnp.dot(hn, wmd_ref[...],
                                preferred_element_type=jnp.float32)
                        + bm_ref[...]).astype(jnp.bfloat16)
        outs[2][...] = jnp.dot(hn, wms_ref[...],
                               preferred_element_type=jnp.float32
                               ).astype(jnp.bfloat16)

        @pl.when(i == 0)
        def _():
            outs[3][...] = jnp.zeros((KP, H), jnp.float32)

        outs[3][...] += lax.dot_general(onehot, hn, (((0,), (0,)), ((), ())),
                                        preferred_element_type=jnp.float32)


def _run_update(last, h, aggr_p, deg_p, csum, ccnt, cl3, rt3, wu, bu,
                wmd, bm, wms):
    whole2 = lambda: pl.BlockSpec(index_map=lambda i: (0, 0))
    out_specs = [pl.BlockSpec((BN, H), lambda i: (i, 0))]
    out_shape = [jax.ShapeDtypeStruct((NP, H), jnp.float32)]
    if not last:
        out_specs += [
            pl.BlockSpec((BN, H), lambda i: (i, 0)),
            pl.BlockSpec((BN, H), lambda i: (i, 0)),
            pl.BlockSpec((KP, H), lambda i: (0, 0)),
        ]
        out_shape += [
            jax.ShapeDtypeStruct((NP, H), jnp.bfloat16),
            jax.ShapeDtypeStruct((NP, H), jnp.bfloat16),
            jax.ShapeDtypeStruct((KP, H), jnp.float32),
        ]
    return pl.pallas_call(
        functools.partial(_update_body, last),
        grid=(NB,),
        in_specs=[
            pl.BlockSpec((BN, H), lambda i: (i, 0)),
            pl.BlockSpec((NC, BN, H), lambda i: (0, i, 0)),
            pl.BlockSpec((NC, BN, H), lambda i: (0, i, 0)),
            whole2(), whole2(),
            pl.BlockSpec((1, 1, BN), lambda i: (i, 0, 0)),
            pl.BlockSpec((1, 1, BN), lambda i: (i, 0, 0)),
            whole2(), whole2(), whole2(), whole2(), whole2(),
        ],
        out_specs=out_specs,
        out_shape=out_shape,
    )(h, aggr_p, deg_p, csum, ccnt, cl3, rt3, wu, bu, wmd, bm, wms)


@functools.cache
def _sc_mesh():
    return plsc.VectorSubcoreMesh(core_axis_name="c", subcore_axis_name="s",
                                  num_cores=NC, num_subcores=NS)


@functools.cache
def _edge_kernel_fn():
    return pl.kernel(
        _edge_body,
        out_type=jax.ShapeDtypeStruct((NC, NP, H), jnp.float32),
        mesh=_sc_mesh(),
        compiler_params=pltpu.CompilerParams(use_tc_tiling_on_sc=False),
        scratch_types=(
            [pltpu.VMEM((CH,), jnp.int32)] * 8
            + [pltpu.VMEM((CH, H // 2), jnp.int32)] * 4
            + [pltpu.VMEM((CH, H), jnp.float32)] * 2
            + [pltpu.SemaphoreType.DMA] * 8
            + [pltpu.VMEM_SHARED((NP, H), jnp.float32)]
        ),
    )


def _edge_body(p_hbm, q_hbm, src_hbm, dst_hbm, out_hbm,
               idxd0, idxd1, idxd2, idxd3, idxs0, idxs1, idxs2, idxs3,
               p0, p1, q0, q1, r0, r1,
               sg0, sg1, si0, si1, si2, si3, ss0, ss1, acc):
    c = lax.axis_index("c")
    s = lax.axis_index("s")
    base = (s * NC + c) * EPW
    idxd = (idxd0, idxd1, idxd2, idxd3)
    idxs = (idxs0, idxs1, idxs2, idxs3)
    pb = (p0, p1)
    qb = (q0, q1)
    rb = (r0, r1)
    sg = (sg0, sg1)
    si = (si0, si1, si2, si3)
    ss = (ss0, ss1)

    def zrow(j, _):
        for k in range(H // 16):
            r0[j, pl.ds(k * 16, 16)] = jnp.zeros((16,), jnp.float32)
        return 0
    lax.fori_loop(0, CH, zrow, 0)

    def zcopy(t, _):
        pltpu.sync_copy(r0, acc.at[pl.ds(s * RPW + t * CH, CH)])
        return 0
    lax.fori_loop(0, RPW // CH, zcopy, 0)
    plsc.subcore_barrier()

    def idx_load(i, slot):
        b = base + i * CH
        pltpu.async_copy(dst_hbm.at[pl.ds(b, CH)], idxd[slot], si[slot])
        pltpu.async_copy(src_hbm.at[pl.ds(b, CH)], idxs[slot], si[slot])

    def wait_idx(slot):
        pltpu.make_async_copy(dst_hbm.at[pl.ds(0, CH)], idxd[slot],
                              si[slot]).wait()
        pltpu.make_async_copy(src_hbm.at[pl.ds(0, CH)], idxs[slot],
                              si[slot]).wait()

    def gather(slot, par):
        pltpu.async_copy(p_hbm.at[idxd[slot]], pb[par], sg[par])
        pltpu.async_copy(q_hbm.at[idxs[slot]], qb[par], sg[par])

    def wait_gather(slot, par):
        pltpu.make_async_copy(p_hbm.at[idxd[slot]], pb[par], sg[par]).wait()
        pltpu.make_async_copy(q_hbm.at[idxs[slot]], qb[par], sg[par]).wait()

    def compute(par):
        hmask = jnp.full((16,), -65536, jnp.int32)
        bc = lambda u: lax.bitcast_convert_type(u, jnp.float32)

        def row(j, _):
            for k in range(H // 32):
                up = pb[par][j, pl.ds(k * 16, 16)]
                uq = qb[par][j, pl.ds(k * 16, 16)]
                va = bc(up << 16) + bc(uq << 16)
                vb = bc(up & hmask) + bc(uq & hmask)
                rb[par][j, pl.ds(k * 32, 16)] = jnp.maximum(va, 0.01 * va)
                rb[par][j, pl.ds(k * 32 + 16, 16)] = jnp.maximum(vb, 0.01 * vb)
            return 0
        lax.fori_loop(0, CH, row, 0)

    def scatter(slot, par):
        pltpu.async_copy(rb[par], acc.at[idxd[slot]], ss[par], add=True)

    def wait_scatter(slot, par):
        pltpu.make_async_copy(rb[par], acc.at[idxd[slot]], ss[par]).wait()

    def stage(i, slot, par, first):
        ns = (slot + 1) % 4
        idx_load(i + 1, ns)
        wait_gather(slot, par)
        compute(par)
        wait_idx(ns)
        if first:
            @pl.when(i >= 1)
            def _():
                wait_scatter((slot + 3) % 4, par ^ 1)
        else:
            wait_scatter((slot + 3) % 4, par ^ 1)
        gather(ns, par ^ 1)
        scatter(slot, par)

    idx_load(0, 0)
    wait_idx(0)
    gather(0, 0)

    def quad(t, _):
        i0 = 4 * t
        stage(i0, 0, 0, True)
        stage(i0 + 1, 1, 1, False)
        stage(i0 + 2, 2, 0, False)
        stage(i0 + 3, 3, 1, False)
        return 0

    lax.fori_loop(0, (NCHUNK - 1) // 4, quad, 0)
    wait_gather(0, 0)
    compute(0)
    wait_scatter(3, 1)
    pltpu.sync_copy(rb[0], acc.at[idxd[0]], add=True)
    plsc.subcore_barrier()
    pltpu.sync_copy(acc.at[pl.ds(s * RPW, RPW)],
                    out_hbm.at[c, pl.ds(s * RPW, RPW)])


def kernel(x, z, edge_index, node_cluster, node_ratio, W_enc, b_enc,
           W_msg, b_msg, W_upd, b_upd):
    steps = W_msg.shape[0]
    xz = jnp.concatenate([x, z], axis=-1)
    xz = jnp.pad(xz, ((0, NP - N), (0, 0)))
    cl = jnp.pad(node_cluster, (0, NP - N), constant_values=KP - 1)
    cl3 = cl.reshape(NB, 1, BN)
    rt = jnp.pad(node_ratio[:, 0], (0, NP - N))
    rt3 = rt.reshape(NB, 1, BN)
    src = edge_index[0]
    dst = edge_index[1]
    b_enc2 = b_enc.reshape(1, H)
    bu2 = b_upd.reshape(steps, 1, H)
    g = jnp.asarray(_PERM)
    wmd = jnp.take(W_msg[:, :H], g, axis=2)
    wms = jnp.take(W_msg[:, H:], g, axis=2)
    bm2 = jnp.take(b_msg, g, axis=1).reshape(steps, 1, H)

    h, p, q, csum, ccnt = _run_encode(
        xz, cl3, W_enc, b_enc2, wmd[0], bm2[0], wms[0])
    pack = lambda a: lax.bitcast_convert_type(
        a.reshape(NP, H // 2, 2), jnp.int32)
    deg_p = _edge_kernel_fn()(pack(jnp.ones((NP, H), jnp.bfloat16)),
                              pack(jnp.zeros((NP, H), jnp.bfloat16)),
                              src, dst)

    for s in range(steps):
        aggr_p = _edge_kernel_fn()(pack(p), pack(q), src, dst)
        last = s == steps - 1
        nxt = min(s + 1, steps - 1)
        outs = _run_update(last, h, aggr_p, deg_p, csum, ccnt, cl3, rt3,
                           W_upd[s], bu2[s], wmd[nxt], bm2[nxt], wms[nxt])
        if last:
            h = outs[0]
        else:
            h, p, q, csum = outs

    return h[:N]


# speedup vs baseline: 1.6266x; 1.6266x over previous
import functools

import jax
import jax.numpy as jnp
import numpy as np
from jax import lax
from jax.experimental import pallas as pl
from jax.experimental.pallas import tpu as pltpu
from jax.experimental.pallas import tpu_sc as plsc

N = 10000
E = 320000
H = 128
KP = 512
BN = 512
NP = 10240
NB = NP // BN

NC = 2
NS = 16
NW = NC * NS
EPW = E // NW
CH = 80
NCHUNK = EPW // CH
assert NCHUNK % 4 == 1
RPW = NP // NS


def _leaky(v):
    return jnp.maximum(v, 0.01 * v)


def _unpack_perm():
    tau = np.empty((H,), np.int64)
    for k in range(H // 32):
        for i in range(16):
            tau[32 * k + i] = 32 * k + 2 * i
            tau[32 * k + 16 + i] = 32 * k + 2 * i + 1
    g = np.empty((H,), np.int64)
    g[tau] = np.arange(H)
    return g


_PERM = _unpack_perm()


def _encode_body(xz_ref, cl_ref, w_enc_ref, b_enc_ref, wmd_ref, bm_ref,
                 wms_ref, h_ref, p_ref, q_ref, csum_ref, ccnt_ref):
    i = pl.program_id(0)
    h = _leaky(jnp.dot(xz_ref[...], w_enc_ref[...],
                       preferred_element_type=jnp.float32) + b_enc_ref[...])
    h_ref[...] = h
    p_ref[...] = jnp.dot(h, wmd_ref[...],
                         preferred_element_type=jnp.float32) + bm_ref[...]
    q_ref[...] = jnp.dot(h, wms_ref[...], preferred_element_type=jnp.float32)
    cl = cl_ref[0, 0, :]
    onehot = (cl[:, None] == lax.broadcasted_iota(jnp.int32, (BN, KP), 1))
    onehot = onehot.astype(jnp.float32)

    @pl.when(i == 0)
    def _():
        csum_ref[...] = jnp.zeros((KP, H), jnp.float32)
        ccnt_ref[...] = jnp.zeros((KP, H), jnp.float32)

    csum_ref[...] += lax.dot_general(onehot, h, (((0,), (0,)), ((), ())),
                                     preferred_element_type=jnp.float32)
    ccnt_ref[...] += jnp.sum(onehot, axis=0)[:, None]


def _run_encode(xz, cl3, w_enc, b_enc, wmd, bm, wms):
    whole2 = lambda: pl.BlockSpec(index_map=lambda i: (0, 0))
    return pl.pallas_call(
        _encode_body,
        grid=(NB,),
        in_specs=[
            pl.BlockSpec((BN, xz.shape[1]), lambda i: (i, 0)),
            pl.BlockSpec((1, 1, BN), lambda i: (i, 0, 0)),
            whole2(), whole2(), whole2(), whole2(), whole2(),
        ],
        out_specs=[
            pl.BlockSpec((BN, H), lambda i: (i, 0)),
            pl.BlockSpec((BN, H), lambda i: (i, 0)),
            pl.BlockSpec((BN, H), lambda i: (i, 0)),
            pl.BlockSpec((KP, H), lambda i: (0, 0)),
            pl.BlockSpec((KP, H), lambda i: (0, 0)),
        ],
        out_shape=[
            jax.ShapeDtypeStruct((NP, H), jnp.float32),
            jax.ShapeDtypeStruct((NP, H), jnp.float32),
            jax.ShapeDtypeStruct((NP, H), jnp.float32),
            jax.ShapeDtypeStruct((KP, H), jnp.float32),
            jax.ShapeDtypeStruct((KP, H), jnp.float32),
        ],
    )(xz, cl3, w_enc, b_enc, wmd, bm, wms)


def _update_body(last, h_ref, ap_ref, dp_ref, csum_ref, ccnt_ref, cl_ref,
                 rt_ref, wu_ref, bu_ref, wmd_ref, bm_ref, wms_ref, *outs):
    i = pl.program_id(0)
    h = h_ref[...]
    d3 = dp_ref[...]
    deg = jnp.maximum(d3[0, :, 0] + d3[1, :, 0], 1.0)
    a3 = ap_ref[...]
    aggr = (a3[0] + a3[1]) / deg[:, None]
    cmean = csum_ref[...] / jnp.maximum(ccnt_ref[...], 1.0)
    cl = cl_ref[0, 0, :]
    onehot = (cl[:, None] == lax.broadcasted_iota(jnp.int32, (BN, KP), 1))
    onehot = onehot.astype(jnp.float32)
    c = jnp.dot(onehot, cmean,
                preferred_element_type=jnp.float32) * rt_ref[0, 0, :][:, None]
    wu = wu_ref[...]
    u = (jnp.dot(h, wu[:H], preferred_element_type=jnp.float32)
         + jnp.dot(aggr, wu[H:2 * H], preferred_element_type=jnp.float32)
         + jnp.dot(c, wu[2 * H:], preferred_element_type=jnp.float32)
         + bu_ref[...])
    hn = h + _leaky(u)
    outs[0][...] = hn
    if not last:
        outs[1][...] = jnp.dot(hn, wmd_ref[...],
                               preferred_element_type=jnp.float32) + bm_ref[...]
        outs[2][...] = jnp.dot(hn, wms_ref[...],
                               preferred_element_type=jnp.float32)

        @pl.when(i == 0)
        def _():
            outs[3][...] = jnp.zeros((KP, H), jnp.float32)

        outs[3][...] += lax.dot_general(onehot, hn, (((0,), (0,)), ((), ())),
                                        preferred_element_type=jnp.float32)


def _run_update(last, h, aggr_p, deg_p, csum, ccnt, cl3, rt3, wu, bu,
                wmd, bm, wms):
    whole2 = lambda: pl.BlockSpec(index_map=lambda i: (0, 0))
    out_specs = [pl.BlockSpec((BN, H), lambda i: (i, 0))]
    out_shape = [jax.ShapeDtypeStruct((NP, H), jnp.float32)]
    if not last:
        out_specs += [
            pl.BlockSpec((BN, H), lambda i: (i, 0)),
            pl.BlockSpec((BN, H), lambda i: (i, 0)),
            pl.BlockSpec((KP, H), lambda i: (0, 0)),
        ]
        out_shape += [
            jax.ShapeDtypeStruct((NP, H), jnp.float32),
            jax.ShapeDtypeStruct((NP, H), jnp.float32),
            jax.ShapeDtypeStruct((KP, H), jnp.float32),
        ]
    return pl.pallas_call(
        functools.partial(_update_body, last),
        grid=(NB,),
        in_specs=[
            pl.BlockSpec((BN, H), lambda i: (i, 0)),
            pl.BlockSpec((NC, BN, H), lambda i: (0, i, 0)),
            pl.BlockSpec((NC, BN, H), lambda i: (0, i, 0)),
            whole2(), whole2(),
            pl.BlockSpec((1, 1, BN), lambda i: (i, 0, 0)),
            pl.BlockSpec((1, 1, BN), lambda i: (i, 0, 0)),
            whole2(), whole2(), whole2(), whole2(), whole2(),
        ],
        out_specs=out_specs,
        out_shape=out_shape,
    )(h, aggr_p, deg_p, csum, ccnt, cl3, rt3, wu, bu, wmd, bm, wms)


@functools.cache
def _sc_mesh():
    return plsc.VectorSubcoreMesh(core_axis_name="c", subcore_axis_name="s",
                                  num_cores=NC, num_subcores=NS)


@functools.cache
def _edge_kernel_fn():
    return pl.kernel(
        _edge_body,
        out_type=jax.ShapeDtypeStruct((NC, NP, H), jnp.float32),
        mesh=_sc_mesh(),
        scratch_types=(
            [pltpu.VMEM((CH,), jnp.int32)] * 8
            + [pltpu.VMEM((CH, H), jnp.float32)] * 4
            + [pltpu.SemaphoreType.DMA] * 8
            + [pltpu.VMEM_SHARED((NP, H), jnp.float32)]
        ),
    )


def _edge_body(p_hbm, q_hbm, src_hbm, dst_hbm, out_hbm,
               idxd0, idxd1, idxd2, idxd3, idxs0, idxs1, idxs2, idxs3,
               p0, p1, q0, q1,
               sg0, sg1, si0, si1, si2, si3, ss0, ss1, acc):
    c = lax.axis_index("c")
    s = lax.axis_index("s")
    base = (s * NC + c) * EPW
    idxd = (idxd0, idxd1, idxd2, idxd3)
    idxs = (idxs0, idxs1, idxs2, idxs3)
    pb = (p0, p1)
    qb = (q0, q1)
    sg = (sg0, sg1)
    si = (si0, si1, si2, si3)
    ss = (ss0, ss1)

    def zrow(j, _):
        for k in range(H // 16):
            q0[j, pl.ds(k * 16, 16)] = jnp.zeros((16,), jnp.float32)
        return 0
    lax.fori_loop(0, CH, zrow, 0)

    def zcopy(t, _):
        pltpu.sync_copy(q0, acc.at[pl.ds(s * RPW + t * CH, CH)])
        return 0
    lax.fori_loop(0, RPW // CH, zcopy, 0)
    plsc.subcore_barrier()

    def idx_load(i, slot):
        b = base + i * CH
        pltpu.async_copy(dst_hbm.at[pl.ds(b, CH)], idxd[slot], si[slot])
        pltpu.async_copy(src_hbm.at[pl.ds(b, CH)], idxs[slot], si[slot])

    def wait_idx(slot):
        pltpu.make_async_copy(dst_hbm.at[pl.ds(0, CH)], idxd[slot],
                              si[slot]).wait()
        pltpu.make_async_copy(src_hbm.at[pl.ds(0, CH)], idxs[slot],
                              si[slot]).wait()

    def gather(slot, par):
        pltpu.async_copy(p_hbm.at[idxd[slot]], pb[par], sg[par])
        pltpu.async_copy(q_hbm.at[idxs[slot]], qb[par], sg[par])

    def wait_gather(slot, par):
        pltpu.make_async_copy(p_hbm.at[idxd[slot]], pb[par], sg[par]).wait()
        pltpu.make_async_copy(q_hbm.at[idxs[slot]], qb[par], sg[par]).wait()

    def compute(par):
        def row(j, _):
            for k in range(H // 16):
                v = (pb[par][j, pl.ds(k * 16, 16)]
                     + qb[par][j, pl.ds(k * 16, 16)])
                pb[par][j, pl.ds(k * 16, 16)] = jnp.maximum(v, 0.01 * v)
            return 0
        lax.fori_loop(0, CH, row, 0)

    def scatter(slot, par):
        pltpu.async_copy(pb[par], acc.at[idxd[slot]], ss[par], add=True)

    def wait_scatter(slot, par):
        pltpu.make_async_copy(pb[par], acc.at[idxd[slot]], ss[par]).wait()

    def stage(i, slot, par, first):
        ns = (slot + 1) % 4
        idx_load(i + 1, ns)
        wait_gather(slot, par)
        compute(par)
        wait_idx(ns)
        if first:
            @pl.when(i >= 1)
            def _():
                wait_scatter((slot + 3) % 4, par ^ 1)
        else:
            wait_scatter((slot + 3) % 4, par ^ 1)
        gather(ns, par ^ 1)
        scatter(slot, par)

    idx_load(0, 0)
    wait_idx(0)
    gather(0, 0)

    def quad(t, _):
        i0 = 4 * t
        stage(i0, 0, 0, True)
        stage(i0 + 1, 1, 1, False)
        stage(i0 + 2, 2, 0, False)
        stage(i0 + 3, 3, 1, False)
        return 0

    lax.fori_loop(0, (NCHUNK - 1) // 4, quad, 0)
    wait_gather(0, 0)
    compute(0)
    wait_scatter(3, 1)
    pltpu.sync_copy(pb[0], acc.at[idxd[0]], add=True)
    plsc.subcore_barrier()
    pltpu.sync_copy(acc.at[pl.ds(s * RPW, RPW)],
                    out_hbm.at[c, pl.ds(s * RPW, RPW)])


def kernel(x, z, edge_index, node_cluster, node_ratio, W_enc, b_enc,
           W_msg, b_msg, W_upd, b_upd):
    steps = W_msg.shape[0]
    xz = jnp.concatenate([x, z], axis=-1)
    xz = jnp.pad(xz, ((0, NP - N), (0, 0)))
    cl = jnp.pad(node_cluster, (0, NP - N), constant_values=KP - 1)
    cl3 = cl.reshape(NB, 1, BN)
    rt = jnp.pad(node_ratio[:, 0], (0, NP - N))
    rt3 = rt.reshape(NB, 1, BN)
    src = edge_index[0]
    dst = edge_index[1]
    b_enc2 = b_enc.reshape(1, H)
    bu2 = b_upd.reshape(steps, 1, H)
    wmd = W_msg[:, :H]
    wms = W_msg[:, H:]
    bm2 = b_msg.reshape(steps, 1, H)

    h, p, q, csum, ccnt = _run_encode(
        xz, cl3, W_enc, b_enc2, wmd[0], bm2[0], wms[0])
    deg_p = _edge_kernel_fn()(jnp.ones((NP, H), jnp.float32),
                              jnp.zeros((NP, H), jnp.float32), src, dst)

    for s in range(steps):
        aggr_p = _edge_kernel_fn()(p, q, src, dst)
        last = s == steps - 1
        nxt = min(s + 1, steps - 1)
        outs = _run_update(last, h, aggr_p, deg_p, csum, ccnt, cl3, rt3,
                           W_upd[s], bu2[s], wmd[nxt], bm2[nxt], wms[nxt])
        if last:
            h = outs[0]
        else:
            h, p, q, csum = outs

    return h[:N]


# speedup vs baseline: 1.8652x; 1.1467x over previous
import functools

import jax
import jax.numpy as jnp
import numpy as np
from jax import lax
from jax.experimental import pallas as pl
from jax.experimental.pallas import tpu as pltpu
from jax.experimental.pallas import tpu_sc as plsc

N = 10000
E = 320000
H = 128
KP = 512
BN = 512
NP = 10240
NB = NP // BN

NC = 2
NS = 16
NW = NC * NS
EPW = E // NW
CH = 80
NCHUNK = EPW // CH
assert NCHUNK % 4 == 1
RPW = NP // NS


def _leaky(v):
    return jnp.maximum(v, 0.01 * v)


def _unpack_perm():
    tau = np.empty((H,), np.int64)
    for k in range(H // 32):
        for i in range(16):
            tau[32 * k + i] = 32 * k + 2 * i
            tau[32 * k + 16 + i] = 32 * k + 2 * i + 1
    g = np.empty((H,), np.int64)
    g[tau] = np.arange(H)
    return g


_PERM = _unpack_perm()


def _encode_body(xz_ref, cl_ref, w_enc_ref, b_enc_ref, wmd_ref, bm_ref,
                 wms_ref, h_ref, p_ref, q_ref, csum_ref, ccnt_ref):
    i = pl.program_id(0)
    h = _leaky(jnp.dot(xz_ref[...], w_enc_ref[...],
                       preferred_element_type=jnp.float32) + b_enc_ref[...])
    h_ref[...] = h
    p_ref[...] = jnp.dot(h, wmd_ref[...],
                         preferred_element_type=jnp.float32) + bm_ref[...]
    q_ref[...] = jnp.dot(h, wms_ref[...], preferred_element_type=jnp.float32)
    cl = cl_ref[0, 0, :]
    onehot = (cl[:, None] == lax.broadcasted_iota(jnp.int32, (BN, KP), 1))
    onehot = onehot.astype(jnp.float32)

    @pl.when(i == 0)
    def _():
        csum_ref[...] = jnp.zeros((KP, H), jnp.float32)
        ccnt_ref[...] = jnp.zeros((KP, H), jnp.float32)

    csum_ref[...] += lax.dot_general(onehot, h, (((0,), (0,)), ((), ())),
                                     preferred_element_type=jnp.float32)
    ccnt_ref[...] += jnp.sum(onehot, axis=0)[:, None]


def _run_encode(xz, cl3, w_enc, b_enc, wmd, bm, wms):
    whole2 = lambda: pl.BlockSpec(index_map=lambda i: (0, 0))
    return pl.pallas_call(
        _encode_body,
        grid=(NB,),
        in_specs=[
            pl.BlockSpec((BN, xz.shape[1]), lambda i: (i, 0)),
            pl.BlockSpec((1, 1, BN), lambda i: (i, 0, 0)),
            whole2(), whole2(), whole2(), whole2(), whole2(),
        ],
        out_specs=[
            pl.BlockSpec((BN, H), lambda i: (i, 0)),
            pl.BlockSpec((BN, H), lambda i: (i, 0)),
            pl.BlockSpec((BN, H), lambda i: (i, 0)),
            pl.BlockSpec((KP, H), lambda i: (0, 0)),
            pl.BlockSpec((KP, H), lambda i: (0, 0)),
        ],
        out_shape=[
            jax.ShapeDtypeStruct((NP, H), jnp.float32),
            jax.ShapeDtypeStruct((NP, H), jnp.float32),
            jax.ShapeDtypeStruct((NP, H), jnp.float32),
            jax.ShapeDtypeStruct((KP, H), jnp.float32),
            jax.ShapeDtypeStruct((KP, H), jnp.float32),
        ],
    )(xz, cl3, w_enc, b_enc, wmd, bm, wms)


def _update_body(last, h_ref, ap_ref, dp_ref, csum_ref, ccnt_ref, cl_ref,
                 rt_ref, wu_ref, bu_ref, wmd_ref, bm_ref, wms_ref, *outs):
    i = pl.program_id(0)
    h = h_ref[...]
    d3 = dp_ref[...]
    deg = jnp.maximum(d3[0, :, 0] + d3[1, :, 0], 1.0)
    a3 = ap_ref[...]
    aggr = (a3[0] + a3[1]) / deg[:, None]
    cmean = csum_ref[...] / jnp.maximum(ccnt_ref[...], 1.0)
    cl = cl_ref[0, 0, :]
    onehot = (cl[:, None] == lax.broadcasted_iota(jnp.int32, (BN, KP), 1))
    onehot = onehot.astype(jnp.float32)
    c = jnp.dot(onehot, cmean,
                preferred_element_type=jnp.float32) * rt_ref[0, 0, :][:, None]
    wu = wu_ref[...]
    u = (jnp.dot(h, wu[:H], preferred_element_type=jnp.float32)
         + jnp.dot(aggr, wu[H:2 * H], preferred_element_type=jnp.float32)
         + jnp.dot(c, wu[2 * H:], preferred_element_type=jnp.float32)
         + bu_ref[...])
    hn = h + _leaky(u)
    outs[0][...] = hn
    if not last:
        outs[1][...] = jnp.dot(hn, wmd_ref[...],
                               preferred_element_type=jnp.float32) + bm_ref[...]
        outs[2][...] = jnp.dot(hn, wms_ref[...],
                               preferred_element_type=jnp.float32)

        @pl.when(i == 0)
        def _():
            outs[3][...] = jnp.zeros((KP, H), jnp.float32)

        outs[3][...] += lax.dot_general(onehot, hn, (((0,), (0,)), ((), ())),
                                        preferred_element_type=jnp.float32)


def _run_update(last, h, aggr_p, deg_p, csum, ccnt, cl3, rt3, wu, bu,
                wmd, bm, wms):
    whole2 = lambda: pl.BlockSpec(index_map=lambda i: (0, 0))
    out_specs = [pl.BlockSpec((BN, H), lambda i: (i, 0))]
    out_shape = [jax.ShapeDtypeStruct((NP, H), jnp.float32)]
    if not last:
        out_specs += [
            pl.BlockSpec((BN, H), lambda i: (i, 0)),
            pl.BlockSpec((BN, H), lambda i: (i, 0)),
            pl.BlockSpec((KP, H), lambda i: (0, 0)),
        ]
        out_shape += [
            jax.ShapeDtypeStruct((NP, H), jnp.float32),
            jax.ShapeDtypeStruct((NP, H), jnp.float32),
            jax.ShapeDtypeStruct((KP, H), jnp.float32),
        ]
    return pl.pallas_call(
        functools.partial(_update_body, last),
        grid=(NB,),
        in_specs=[
            pl.BlockSpec((BN, H), lambda i: (i, 0)),
            pl.BlockSpec((NC, BN, H), lambda i: (0, i, 0)),
            pl.BlockSpec((NC, BN, H), lambda i: (0, i, 0)),
            whole2(), whole2(),
            pl.BlockSpec((1, 1, BN), lambda i: (i, 0, 0)),
            pl.BlockSpec((1, 1, BN), lambda i: (i, 0, 0)),
            whole2(), whole2(), whole2(), whole2(), whole2(),
        ],
        out_specs=out_specs,
        out_shape=out_shape,
    )(h, aggr_p, deg_p, csum, ccnt, cl3, rt3, wu, bu, wmd, bm, wms)


@functools.cache
def _sc_mesh():
    return plsc.VectorSubcoreMesh(core_axis_name="c", subcore_axis_name="s",
                                  num_cores=NC, num_subcores=NS)


@functools.cache
def _edge_kernel_fn():
    return pl.kernel(
        _edge_body,
        out_type=jax.ShapeDtypeStruct((NC, NP, H), jnp.float32),
        mesh=_sc_mesh(),
        scratch_types=(
            [pltpu.VMEM((16,), jnp.int32)]
            + [pltpu.VMEM((CH,), jnp.int32)] * 8
            + [pltpu.VMEM((CH, H), jnp.float32)] * 4
            + [pltpu.SemaphoreType.DMA] * 8
            + [pltpu.VMEM_SHARED((NP, H), jnp.float32)]
        ),
    )


def _edge_body(mode_hbm, p_hbm, q_hbm, src_hbm, dst_hbm, out_hbm,
               mflag, idxd0, idxd1, idxd2, idxd3, idxs0, idxs1, idxs2, idxs3,
               p0, p1, q0, q1,
               sg0, sg1, si0, si1, si2, si3, ss0, ss1, acc):
    c = lax.axis_index("c")
    s = lax.axis_index("s")
    base = (s * NC + c) * EPW
    idxd = (idxd0, idxd1, idxd2, idxd3)
    idxs = (idxs0, idxs1, idxs2, idxs3)
    pb = (p0, p1)
    qb = (q0, q1)
    sg = (sg0, sg1)
    si = (si0, si1, si2, si3)
    ss = (ss0, ss1)

    def zrow(j, _):
        for k in range(H // 16):
            q0[j, pl.ds(k * 16, 16)] = jnp.zeros((16,), jnp.float32)
        return 0
    lax.fori_loop(0, CH, zrow, 0)

    def zcopy(t, _):
        pltpu.sync_copy(q0, acc.at[pl.ds(s * RPW + t * CH, CH)])
        return 0
    lax.fori_loop(0, RPW // CH, zcopy, 0)
    pltpu.sync_copy(mode_hbm, mflag)
    deg_mode = mflag[...][0] > 0
    plsc.subcore_barrier()

    def idx_load(i, slot):
        b = base + i * CH
        pltpu.async_copy(dst_hbm.at[pl.ds(b, CH)], idxd[slot], si[slot])
        pltpu.async_copy(src_hbm.at[pl.ds(b, CH)], idxs[slot], si[slot])

    def wait_idx(slot):
        pltpu.make_async_copy(dst_hbm.at[pl.ds(0, CH)], idxd[slot],
                              si[slot]).wait()
        pltpu.make_async_copy(src_hbm.at[pl.ds(0, CH)], idxs[slot],
                              si[slot]).wait()

    def gather(slot, par):
        pltpu.async_copy(p_hbm.at[idxd[slot]], pb[par], sg[par])
        pltpu.async_copy(q_hbm.at[idxs[slot]], qb[par], sg[par])

    def wait_gather(slot, par):
        pltpu.make_async_copy(p_hbm.at[idxd[slot]], pb[par], sg[par]).wait()
        pltpu.make_async_copy(q_hbm.at[idxs[slot]], qb[par], sg[par]).wait()

    def compute(par):
        def row(j, _):
            for k in range(H // 16):
                v = (pb[par][j, pl.ds(k * 16, 16)]
                     + qb[par][j, pl.ds(k * 16, 16)])
                pb[par][j, pl.ds(k * 16, 16)] = jnp.maximum(v, 0.01 * v)
            return 0
        lax.fori_loop(0, CH, row, 0)

    def scatter(slot, par):
        pltpu.async_copy(pb[par], acc.at[idxd[slot]], ss[par], add=True)

    def wait_scatter(slot, par):
        pltpu.make_async_copy(pb[par], acc.at[idxd[slot]], ss[par]).wait()

    def stage(i, slot, par, first):
        ns = (slot + 1) % 4
        idx_load(i + 1, ns)
        wait_gather(slot, par)
        compute(par)
        wait_idx(ns)
        if first:
            @pl.when(i >= 1)
            def _():
                wait_scatter((slot + 3) % 4, par ^ 1)
        else:
            wait_scatter((slot + 3) % 4, par ^ 1)
        gather(ns, par ^ 1)
        scatter(slot, par)

    @pl.when(deg_mode)
    def _():
        def orow(j, _):
            for k in range(H // 16):
                p0[j, pl.ds(k * 16, 16)] = jnp.ones((16,), jnp.float32)
            return 0
        lax.fori_loop(0, CH, orow, 0)

        def dchunk(i, _):
            pltpu.sync_copy(dst_hbm.at[pl.ds(base + i * CH, CH)], idxd0)
            pltpu.sync_copy(p0, acc.at[idxd0], add=True)
            return 0
        lax.fori_loop(0, NCHUNK, dchunk, 0)

    @pl.when(jnp.logical_not(deg_mode))
    def _():
        idx_load(0, 0)
        wait_idx(0)
        gather(0, 0)

        def quad(t, _):
            i0 = 4 * t
            stage(i0, 0, 0, True)
            stage(i0 + 1, 1, 1, False)
            stage(i0 + 2, 2, 0, False)
            stage(i0 + 3, 3, 1, False)
            return 0

        lax.fori_loop(0, (NCHUNK - 1) // 4, quad, 0)
        wait_gather(0, 0)
        compute(0)
        wait_scatter(3, 1)
        pltpu.sync_copy(pb[0], acc.at[idxd[0]], add=True)

    plsc.subcore_barrier()
    pltpu.sync_copy(acc.at[pl.ds(s * RPW, RPW)],
                    out_hbm.at[c, pl.ds(s * RPW, RPW)])


def kernel(x, z, edge_index, node_cluster, node_ratio, W_enc, b_enc,
           W_msg, b_msg, W_upd, b_upd):
    steps = W_msg.shape[0]
    xz = jnp.concatenate([x, z], axis=-1)
    xz = jnp.pad(xz, ((0, NP - N), (0, 0)))
    cl = jnp.pad(node_cluster, (0, NP - N), constant_values=KP - 1)
    cl3 = cl.reshape(NB, 1, BN)
    rt = jnp.pad(node_ratio[:, 0], (0, NP - N))
    rt3 = rt.reshape(NB, 1, BN)
    src = edge_index[0]
    dst = edge_index[1]
    b_enc2 = b_enc.reshape(1, H)
    bu2 = b_upd.reshape(steps, 1, H)
    wmd = W_msg[:, :H]
    wms = W_msg[:, H:]
    bm2 = b_msg.reshape(steps, 1, H)

    h, p, q, csum, ccnt = _run_encode(
        xz, cl3, W_enc, b_enc2, wmd[0], bm2[0], wms[0])
    mode0 = jnp.zeros((16,), jnp.int32)
    mode1 = jnp.ones((16,), jnp.int32)
    deg_p = _edge_kernel_fn()(mode1, p, q, src, dst)

    for s in range(steps):
        aggr_p = _edge_kernel_fn()(mode0, p, q, src, dst)
        last = s == steps - 1
        nxt = min(s + 1, steps - 1)
        outs = _run_update(last, h, aggr_p, deg_p, csum, ccnt, cl3, rt3,
                           W_upd[s], bu2[s], wmd[nxt], bm2[nxt], wms[nxt])
        if last:
            h = outs[0]
        else:
            h, p, q, csum = outs

    return h[:N]


# speedup vs baseline: 1.9502x; 1.0455x over previous
import functools

import jax
import jax.numpy as jnp
import numpy as np
from jax import lax
from jax.experimental import pallas as pl
from jax.experimental.pallas import tpu as pltpu
from jax.experimental.pallas import tpu_sc as plsc

N = 10000
E = 320000
H = 128
KP = 512
BN = 512
NP = 10240
NB = NP // BN

NC = 2
NS = 16
NW = NC * NS
EPW = E // NW
CH = 80
NCHUNK = EPW // CH
assert NCHUNK % 4 == 1
RPW = NP // NS


def _leaky(v):
    return jnp.maximum(v, 0.01 * v)


def _unpack_perm():
    tau = np.empty((H,), np.int64)
    for k in range(H // 32):
        for i in range(16):
            tau[32 * k + i] = 32 * k + 2 * i
            tau[32 * k + 16 + i] = 32 * k + 2 * i + 1
    g = np.empty((H,), np.int64)
    g[tau] = np.arange(H)
    return g


_PERM = _unpack_perm()


def _encode_body(xz_ref, cl_ref, w_enc_ref, b_enc_ref, wmd_ref, bm_ref,
                 wms_ref, h_ref, p_ref, q_ref, csum_ref, ccnt_ref):
    i = pl.program_id(0)
    h = _leaky(jnp.dot(xz_ref[...], w_enc_ref[...],
                       preferred_element_type=jnp.float32) + b_enc_ref[...])
    h_ref[...] = h
    p_ref[...] = jnp.dot(h, wmd_ref[...],
                         preferred_element_type=jnp.float32) + bm_ref[...]
    q_ref[...] = jnp.dot(h, wms_ref[...], preferred_element_type=jnp.float32)
    cl = cl_ref[0, 0, :]
    onehot = (cl[:, None] == lax.broadcasted_iota(jnp.int32, (BN, KP), 1))
    onehot = onehot.astype(jnp.float32)

    @pl.when(i == 0)
    def _():
        csum_ref[...] = jnp.zeros((KP, H), jnp.float32)
        ccnt_ref[...] = jnp.zeros((KP, H), jnp.float32)

    csum_ref[...] += lax.dot_general(onehot, h, (((0,), (0,)), ((), ())),
                                     preferred_element_type=jnp.float32)
    ccnt_ref[...] += jnp.sum(onehot, axis=0)[:, None]


def _run_encode(xz, cl3, w_enc, b_enc, wmd, bm, wms):
    whole2 = lambda: pl.BlockSpec(index_map=lambda i: (0, 0))
    return pl.pallas_call(
        _encode_body,
        grid=(NB,),
        in_specs=[
            pl.BlockSpec((BN, xz.shape[1]), lambda i: (i, 0)),
            pl.BlockSpec((1, 1, BN), lambda i: (i, 0, 0)),
            whole2(), whole2(), whole2(), whole2(), whole2(),
        ],
        out_specs=[
            pl.BlockSpec((BN, H), lambda i: (i, 0)),
            pl.BlockSpec((BN, H), lambda i: (i, 0)),
            pl.BlockSpec((BN, H), lambda i: (i, 0)),
            pl.BlockSpec((KP, H), lambda i: (0, 0)),
            pl.BlockSpec((KP, H), lambda i: (0, 0)),
        ],
        out_shape=[
            jax.ShapeDtypeStruct((NP, H), jnp.float32),
            jax.ShapeDtypeStruct((NP, H), jnp.float32),
            jax.ShapeDtypeStruct((NP, H), jnp.float32),
            jax.ShapeDtypeStruct((KP, H), jnp.float32),
            jax.ShapeDtypeStruct((KP, H), jnp.float32),
        ],
    )(xz, cl3, w_enc, b_enc, wmd, bm, wms)


def _update_body(last, h_ref, ap_ref, dp_ref, csum_ref, ccnt_ref, cl_ref,
                 rt_ref, wu_ref, bu_ref, wmd_ref, bm_ref, wms_ref, *outs):
    i = pl.program_id(0)
    h = h_ref[...]
    d3 = dp_ref[...]
    deg = jnp.maximum(d3[0, :, 0] + d3[1, :, 0], 1.0)
    a3 = ap_ref[...]
    aggr = (a3[0] + a3[1]) / deg[:, None]
    cmean = csum_ref[...] / jnp.maximum(ccnt_ref[...], 1.0)
    cl = cl_ref[0, 0, :]
    onehot = (cl[:, None] == lax.broadcasted_iota(jnp.int32, (BN, KP), 1))
    onehot = onehot.astype(jnp.float32)
    c = jnp.dot(onehot, cmean,
                preferred_element_type=jnp.float32) * rt_ref[0, 0, :][:, None]
    wu = wu_ref[...]
    u = (jnp.dot(h, wu[:H], preferred_element_type=jnp.float32)
         + jnp.dot(aggr, wu[H:2 * H], preferred_element_type=jnp.float32)
         + jnp.dot(c, wu[2 * H:], preferred_element_type=jnp.float32)
         + bu_ref[...])
    hn = h + _leaky(u)
    outs[0][...] = hn
    if not last:
        outs[1][...] = jnp.dot(hn, wmd_ref[...],
                               preferred_element_type=jnp.float32) + bm_ref[...]
        outs[2][...] = jnp.dot(hn, wms_ref[...],
                               preferred_element_type=jnp.float32)

        @pl.when(i == 0)
        def _():
            outs[3][...] = jnp.zeros((KP, H), jnp.float32)

        outs[3][...] += lax.dot_general(onehot, hn, (((0,), (0,)), ((), ())),
                                        preferred_element_type=jnp.float32)


def _run_update(last, h, aggr_p, deg_p, csum, ccnt, cl3, rt3, wu, bu,
                wmd, bm, wms):
    whole2 = lambda: pl.BlockSpec(index_map=lambda i: (0, 0))
    out_specs = [pl.BlockSpec((BN, H), lambda i: (i, 0))]
    out_shape = [jax.ShapeDtypeStruct((NP, H), jnp.float32)]
    if not last:
        out_specs += [
            pl.BlockSpec((BN, H), lambda i: (i, 0)),
            pl.BlockSpec((BN, H), lambda i: (i, 0)),
            pl.BlockSpec((KP, H), lambda i: (0, 0)),
        ]
        out_shape += [
            jax.ShapeDtypeStruct((NP, H), jnp.float32),
            jax.ShapeDtypeStruct((NP, H), jnp.float32),
            jax.ShapeDtypeStruct((KP, H), jnp.float32),
        ]
    return pl.pallas_call(
        functools.partial(_update_body, last),
        grid=(NB,),
        in_specs=[
            pl.BlockSpec((BN, H), lambda i: (i, 0)),
            pl.BlockSpec((NC, BN, H), lambda i: (0, i, 0)),
            pl.BlockSpec((NC, BN, H), lambda i: (0, i, 0)),
            whole2(), whole2(),
            pl.BlockSpec((1, 1, BN), lambda i: (i, 0, 0)),
            pl.BlockSpec((1, 1, BN), lambda i: (i, 0, 0)),
            whole2(), whole2(), whole2(), whole2(), whole2(),
        ],
        out_specs=out_specs,
        out_shape=out_shape,
    )(h, aggr_p, deg_p, csum, ccnt, cl3, rt3, wu, bu, wmd, bm, wms)


@functools.cache
def _sc_mesh():
    return plsc.VectorSubcoreMesh(core_axis_name="c", subcore_axis_name="s",
                                  num_cores=NC, num_subcores=NS)


@functools.cache
def _edge_kernel_fn():
    return pl.kernel(
        _edge_body,
        out_type=jax.ShapeDtypeStruct((NC, NP, H), jnp.float32),
        mesh=_sc_mesh(),
        scratch_types=(
            [pltpu.VMEM((16,), jnp.int32)]
            + [pltpu.VMEM((CH,), jnp.int32)] * 8
            + [pltpu.VMEM((CH, H), jnp.float32)] * 4
            + [pltpu.SemaphoreType.DMA] * 8
            + [pltpu.VMEM_SHARED((NP, H), jnp.float32)]
        ),
    )


def _edge_body(mode_hbm, p_hbm, q_hbm, src_hbm, dst_hbm, out_hbm,
               mflag, idxd0, idxd1, idxd2, idxd3, idxs0, idxs1, idxs2, idxs3,
               p0, p1, q0, q1,
               sg0, sg1, si0, si1, si2, si3, ss0, ss1, acc):
    c = lax.axis_index("c")
    s = lax.axis_index("s")
    base = (s * NC + c) * EPW
    idxd = (idxd0, idxd1, idxd2, idxd3)
    idxs = (idxs0, idxs1, idxs2, idxs3)
    pb = (p0, p1)
    qb = (q0, q1)
    sg = (sg0, sg1)
    si = (si0, si1, si2, si3)
    ss = (ss0, ss1)

    def zrow(j, _):
        for k in range(H // 16):
            q0[j, pl.ds(k * 16, 16)] = jnp.zeros((16,), jnp.float32)
        return 0
    lax.fori_loop(0, CH, zrow, 0)

    def zcopy(t, _):
        pltpu.sync_copy(q0, acc.at[pl.ds(s * RPW + t * CH, CH)])
        return 0
    lax.fori_loop(0, RPW // CH, zcopy, 0)
    pltpu.sync_copy(mode_hbm, mflag)
    deg_mode = mflag[...][0] > 0
    plsc.subcore_barrier()

    def idx_load(i, slot):
        b = base + i * CH
        pltpu.async_copy(dst_hbm.at[pl.ds(b, CH)], idxd[slot], si[slot])
        pltpu.async_copy(src_hbm.at[pl.ds(b, CH)], idxs[slot], si[slot])

    def wait_idx(slot):
        pltpu.make_async_copy(dst_hbm.at[pl.ds(0, CH)], idxd[slot],
                              si[slot]).wait()
        pltpu.make_async_copy(src_hbm.at[pl.ds(0, CH)], idxs[slot],
                              si[slot]).wait()

    def gather(slot, par):
        pltpu.async_copy(p_hbm.at[idxd[slot]], pb[par], sg[par])
        pltpu.async_copy(q_hbm.at[idxs[slot]], qb[par], sg[par])

    def wait_gather(slot, par):
        pltpu.make_async_copy(p_hbm.at[idxd[slot]], pb[par], sg[par]).wait()
        pltpu.make_async_copy(q_hbm.at[idxs[slot]], qb[par], sg[par]).wait()

    def compute(par):
        def row(j, _):
            for k in range(H // 16):
                v = (pb[par][j, pl.ds(k * 16, 16)]
                     + qb[par][j, pl.ds(k * 16, 16)])
                pb[par][j, pl.ds(k * 16, 16)] = jnp.maximum(v, 0.01 * v)
            return 0
        lax.fori_loop(0, CH, row, 0)

    def scatter(slot, par):
        pltpu.async_copy(pb[par], acc.at[idxd[slot]], ss[par], add=True)

    def wait_scatter(slot, par):
        pltpu.make_async_copy(pb[par], acc.at[idxd[slot]], ss[par]).wait()

    def stage(i, slot, par, first):
        ns = (slot + 1) % 4
        idx_load(i + 1, ns)
        wait_gather(slot, par)
        compute(par)
        wait_idx(ns)
        if first:
            @pl.when(i >= 1)
            def _():
                wait_scatter((slot + 3) % 4, par ^ 1)
        else:
            wait_scatter((slot + 3) % 4, par ^ 1)
        gather(ns, par ^ 1)
        scatter(slot, par)

    @pl.when(deg_mode)
    def _():
        def orow(j, _):
            for k in range(H // 16):
                p0[j, pl.ds(k * 16, 16)] = jnp.ones((16,), jnp.float32)
            return 0
        lax.fori_loop(0, CH, orow, 0)

        def dload(i, slot):
            pltpu.async_copy(dst_hbm.at[pl.ds(base + i * CH, CH)],
                             idxd[slot], si[slot])

        def dwait(slot):
            pltpu.make_async_copy(dst_hbm.at[pl.ds(0, CH)], idxd[slot],
                                  si[slot]).wait()

        def dscat(slot):
            pltpu.sync_copy(p0, acc.at[idxd[slot]], add=True)

        dload(0, 0)

        def dpair(t, _):
            i0 = 2 * t
            dload(i0 + 1, 1)
            dwait(0)
            dscat(0)
            dload(i0 + 2, 0)
            dwait(1)
            dscat(1)
            return 0

        lax.fori_loop(0, (NCHUNK - 1) // 2, dpair, 0)
        dwait(0)
        dscat(0)

    @pl.when(jnp.logical_not(deg_mode))
    def _():
        idx_load(0, 0)
        wait_idx(0)
        gather(0, 0)

        def quad(t, _):
            i0 = 4 * t
            stage(i0, 0, 0, True)
            stage(i0 + 1, 1, 1, False)
            stage(i0 + 2, 2, 0, False)
            stage(i0 + 3, 3, 1, False)
            return 0

        lax.fori_loop(0, (NCHUNK - 1) // 4, quad, 0)
        wait_gather(0, 0)
        compute(0)
        wait_scatter(3, 1)
        pltpu.sync_copy(pb[0], acc.at[idxd[0]], add=True)

    plsc.subcore_barrier()
    pltpu.sync_copy(acc.at[pl.ds(s * RPW, RPW)],
                    out_hbm.at[c, pl.ds(s * RPW, RPW)])


def kernel(x, z, edge_index, node_cluster, node_ratio, W_enc, b_enc,
           W_msg, b_msg, W_upd, b_upd):
    steps = W_msg.shape[0]
    xz = jnp.concatenate([x, z], axis=-1)
    xz = jnp.pad(xz, ((0, NP - N), (0, 0)))
    cl = jnp.pad(node_cluster, (0, NP - N), constant_values=KP - 1)
    cl3 = cl.reshape(NB, 1, BN)
    rt = jnp.pad(node_ratio[:, 0], (0, NP - N))
    rt3 = rt.reshape(NB, 1, BN)
    src = edge_index[0]
    dst = edge_index[1]
    b_enc2 = b_enc.reshape(1, H)
    bu2 = b_upd.reshape(steps, 1, H)
    wmd = W_msg[:, :H]
    wms = W_msg[:, H:]
    bm2 = b_msg.reshape(steps, 1, H)

    h, p, q, csum, ccnt = _run_encode(
        xz, cl3, W_enc, b_enc2, wmd[0], bm2[0], wms[0])
    mode0 = jnp.zeros((16,), jnp.int32)
    mode1 = jnp.ones((16,), jnp.int32)
    deg_p = _edge_kernel_fn()(mode1, p, q, src, dst)

    for s in range(steps):
        aggr_p = _edge_kernel_fn()(mode0, p, q, src, dst)
        last = s == steps - 1
        nxt = min(s + 1, steps - 1)
        outs = _run_update(last, h, aggr_p, deg_p, csum, ccnt, cl3, rt3,
                           W_upd[s], bu2[s], wmd[nxt], bm2[nxt], wms[nxt])
        if last:
            h = outs[0]
        else:
            h, p, q, csum = outs

    return h[:N]
